# baseline (speedup 1.0000x reference)
import functools

import jax
import jax.numpy as jnp
from jax import lax
from jax.experimental import pallas as pl
from jax.experimental.pallas import tpu as pltpu
from jax.experimental.pallas import tpu_sc as plsc

VOCAB = 100000
EMBED_DIM = 64
BATCH = 4096
HIST = 50

_NW = 32
_NB = BATCH // _NW


@functools.partial(
    pl.kernel,
    mesh=plsc.VectorSubcoreMesh(core_axis_name="c", subcore_axis_name="s"),
    out_type=jax.ShapeDtypeStruct((HIST, 8, _NW, 8, _NB), jnp.float32),
    scratch_types=[
        pltpu.VMEM((HIST, _NB), jnp.int32),
        [pltpu.VMEM((_NB, EMBED_DIM), jnp.float32) for _ in range(2)],
        [pltpu.VMEM((8, 8, _NB + 1), jnp.float32) for _ in range(2)],
        [pltpu.SemaphoreType.DMA for _ in range(2)],
        [pltpu.SemaphoreType.DMA for _ in range(2)],
    ],
    compiler_params=pltpu.CompilerParams(use_tc_tiling_on_sc=False, needs_layout_passes=False),
)
def _gather_kernel(idx_hbm, table_hbm, out_hbm, idx_v, rows, trs, g_sems, s_sems):
    c = lax.axis_index("c")
    s = lax.axis_index("s")
    wid = s * 2 + c
    pltpu.sync_copy(idx_hbm.at[wid], idx_v)

    def g_copy(t, b):
        return pltpu.make_async_copy(table_hbm.at[idx_v.at[t]], rows[b], g_sems[b])

    def s_copy(t, b):
        return pltpu.make_async_copy(
            trs[b].at[:, :, pl.ds(0, _NB)],
            out_hbm.at[t, :, wid, :, :],
            s_sems[b],
        )

    def s_start(t, b):
        s_copy(t, b).start()

    def s_wait(t, b):
        s_copy(t, b).wait()

    lane = lax.iota(jnp.int32, 16)
    evecs = [lane + e16 * 16 for e16 in range(EMBED_DIM // 16)]
    e8vecs = [ev // 8 for ev in evecs]
    e0vecs = [ev % 8 for ev in evecs]

    def transpose_block(b):
        rows_ref, tr_ref = rows[b], trs[b]

        @plsc.parallel_loop(0, _NB, unroll=16)
        def b0_body(b0):
            col = jnp.broadcast_to(b0, (16,)).astype(jnp.int32)
            for e16 in range(EMBED_DIM // 16):
                v = rows_ref[b0, pl.ds(e16 * 16, 16)]
                plsc.store_scatter(tr_ref, [e8vecs[e16], e0vecs[e16], col], v)

    g_copy(0, 0).start()
    g_copy(1, 1).start()
    for t in range(2):
        b = t % 2
        g_copy(t, b).wait()
        transpose_block(b)
        g_copy(t + 2, b).start()
        s_start(t, b)

    def group(g, carry):
        for b in range(2):
            t = g * 2 + b
            g_copy(t, b).wait()
            s_wait(t, b)
            transpose_block(b)
            g_copy(t + 2, b).start()
            s_start(t, b)
        return carry

    lax.fori_loop(1, HIST // 2 - 1, group, 0)

    for t in range(HIST - 2, HIST):
        b = t % 2
        g_copy(t, b).wait()
        s_wait(t, b)
        transpose_block(b)
        s_start(t, b)
    for b in range(2):
        s_wait(b, b)


def kernel(x, vectors):
    idx = x.astype(jnp.int32).reshape(_NW, _NB, HIST).transpose(0, 2, 1)
    out = _gather_kernel(idx, vectors)
    return out.transpose(2, 4, 0, 1, 3).reshape(BATCH, HIST, EMBED_DIM)


# speedup vs baseline: 1.0084x; 1.0084x over previous
import functools

import jax
import jax.numpy as jnp
from jax import lax
from jax.experimental import pallas as pl
from jax.experimental.pallas import tpu as pltpu
from jax.experimental.pallas import tpu_sc as plsc

VOCAB = 100000
EMBED_DIM = 64
BATCH = 4096
HIST = 50

_NW = 32
_NB = BATCH // _NW


@functools.partial(
    pl.kernel,
    mesh=plsc.VectorSubcoreMesh(core_axis_name="c", subcore_axis_name="s"),
    out_type=jax.ShapeDtypeStruct((HIST, 8, _NW, 8, _NB), jnp.float32),
    scratch_types=[
        pltpu.VMEM((HIST, _NB), jnp.int32),
        [pltpu.VMEM((_NB, EMBED_DIM), jnp.float32) for _ in range(2)],
        [pltpu.VMEM((EMBED_DIM, _NB + 1), jnp.float32) for _ in range(2)],
        [pltpu.SemaphoreType.DMA for _ in range(2)],
        [pltpu.SemaphoreType.DMA for _ in range(2)],
    ],
    compiler_params=pltpu.CompilerParams(use_tc_tiling_on_sc=False, needs_layout_passes=False),
)
def _gather_kernel(idx_hbm, table_hbm, out_hbm, idx_v, rows, trs, g_sems, s_sems):
    c = lax.axis_index("c")
    s = lax.axis_index("s")
    wid = s * 2 + c
    pltpu.sync_copy(idx_hbm.at[wid], idx_v)

    def g_copy(t, b):
        return pltpu.make_async_copy(table_hbm.at[idx_v.at[t]], rows[b], g_sems[b])

    def _s_copies(t, b):
        return [
            pltpu.make_async_copy(
                trs[b].at[pl.ds(e8 * 8, 8), pl.ds(0, _NB)],
                out_hbm.at[t, e8, wid, :, :],
                s_sems[b],
            )
            for e8 in range(8)
        ]

    def s_start(t, b):
        for cp in _s_copies(t, b):
            cp.start()

    def s_wait(t, b):
        for cp in _s_copies(t, b):
            cp.wait()

    lane = lax.iota(jnp.int32, 16)
    evecs = [lane + e16 * 16 for e16 in range(EMBED_DIM // 16)]

    def transpose_block(b):
        rows_ref, tr_ref = rows[b], trs[b]

        @plsc.parallel_loop(0, _NB, unroll=8)
        def b0_body(b0):
            col = jnp.broadcast_to(b0, (16,)).astype(jnp.int32)
            for e16 in range(EMBED_DIM // 16):
                v = rows_ref[b0, pl.ds(e16 * 16, 16)]
                plsc.store_scatter(tr_ref, [evecs[e16], col], v)

    g_copy(0, 0).start()
    g_copy(1, 1).start()
    for t in range(2):
        b = t % 2
        g_copy(t, b).wait()
        transpose_block(b)
        g_copy(t + 2, b).start()
        s_start(t, b)

    def group(g, carry):
        for b in range(2):
            t = g * 2 + b
            g_copy(t, b).wait()
            s_wait(t, b)
            transpose_block(b)
            g_copy(t + 2, b).start()
            s_start(t, b)
        return carry

    lax.fori_loop(1, HIST // 2 - 1, group, 0)

    for t in range(HIST - 2, HIST):
        b = t % 2
        g_copy(t, b).wait()
        s_wait(t, b)
        transpose_block(b)
        s_start(t, b)
    for b in range(2):
        s_wait(b, b)


def kernel(x, vectors):
    idx = x.astype(jnp.int32).reshape(_NW, _NB, HIST).transpose(0, 2, 1)
    out = _gather_kernel(idx, vectors)
    return out.transpose(2, 4, 0, 1, 3).reshape(BATCH, HIST, EMBED_DIM)
